# baseline (speedup 1.0000x reference)
import functools

import jax
import jax.numpy as jnp
from jax import lax
from jax.experimental import pallas as pl
from jax.experimental.pallas import tpu as pltpu
from jax.experimental.pallas import tpu_sc as plsc

NC = 2
NS = 16
NW = NC * NS
LANES = 16
EB = 128
PF = 4
RB = 1280
F32 = jnp.float32
HIGH = lax.Precision.HIGHEST


def _mesh():
    return plsc.VectorSubcoreMesh(
        core_axis_name="c", subcore_axis_name="s", num_cores=NC, num_subcores=NS
    )


def _sc_deg_body(dst3_hbm, out_hbm, idx_v, deg_v):
    c = lax.axis_index("c")
    s = lax.axis_index("s")
    wid = c * NS + s
    npad = deg_v.shape[0]
    nblk = dst3_hbm.shape[1]

    zeros16 = jnp.zeros((LANES,), F32)
    def zero_body(i, carry):
        deg_v[pl.ds(i * LANES, LANES)] = zeros16
        return carry
    lax.fori_loop(0, npad // LANES, zero_body, 0)

    pltpu.sync_copy(dst3_hbm.at[wid], idx_v)
    ones16 = jnp.ones((LANES,), F32)

    def body(j, carry):
        for k in range(EB // LANES):
            d = idx_v[j, pl.ds(k * LANES, LANES)]
            plsc.addupdate_scatter(deg_v, [d], ones16)
        return carry
    lax.fori_loop(0, nblk, body, 0)

    pltpu.sync_copy(deg_v, out_hbm.at[wid])


def _sc_agg_body(m_hbm, ei5_hbm, out_hbm, pa_v, pb_v, rows_a, rows_b,
                 sem_pa, sem_pb, sem_a0, sem_a1, sem_b0, sem_b1, acc_sh):
    c = lax.axis_index("c")
    s = lax.axis_index("s")
    wid = c * NS + s
    npad = m_hbm.shape[0]
    npanel = ei5_hbm.shape[1]
    rpt = npad // NS
    row0 = s * rpt

    zeros16 = jnp.zeros((LANES,), F32)
    def zero_body(i, carry):
        for k in range(128 // LANES):
            rows_a[i, pl.ds(k * LANES, LANES)] = zeros16
        return carry
    lax.fori_loop(0, EB, zero_body, 0)
    for q in range(rpt // EB):
        pltpu.sync_copy(rows_a, acc_sh.at[pl.ds(row0 + q * EB, EB)])
    plsc.subcore_barrier()

    slots = (rows_a, rows_b)
    sems = ((sem_a0, sem_a1), (sem_b0, sem_b1))
    HB = EB // 2

    def start_gather(panel, b, sl):
        for hh in range(2):
            pltpu.async_copy(
                m_hbm.at[panel.at[b, 0, pl.ds(hh * HB, HB)]],
                slots[sl].at[pl.ds(hh * HB, HB)], sems[sl][hh])

    def wait_scat(panel, b, sl):
        for hh in range(2):
            pltpu.make_async_copy(
                m_hbm.at[panel.at[b, 0, pl.ds(hh * HB, HB)]],
                slots[sl].at[pl.ds(hh * HB, HB)], sems[sl][hh]).wait()
        pltpu.sync_copy(slots[sl], acc_sh.at[panel.at[b, 1]], add=True)

    def load_panel(p_idx, panel, sem):
        pltpu.async_copy(ei5_hbm.at[wid, p_idx], panel, sem)

    def wait_panel(panel, sem):
        pltpu.make_async_copy(ei5_hbm.at[wid, 0], panel, sem).wait()

    pltpu.sync_copy(ei5_hbm.at[wid, 0], pa_v)
    start_gather(pa_v, 0, 0)
    load_panel(1, pb_v, sem_pb)

    npq = npanel // 2

    def body(q, carry):
        for b in range(PF):
            if b < PF - 1:
                start_gather(pa_v, b + 1, (b + 1) % 2)
            else:
                wait_panel(pb_v, sem_pb)
                start_gather(pb_v, 0, 0)
            wait_scat(pa_v, b, b % 2)

        @pl.when(q < npq - 1)
        def _():
            load_panel(2 * q + 2, pa_v, sem_pa)

        for b in range(PF):
            if b < PF - 1:
                start_gather(pb_v, b + 1, (b + 1) % 2)
            else:
                @pl.when(q < npq - 1)
                def _():
                    wait_panel(pa_v, sem_pa)
                    start_gather(pa_v, 0, 0)
            wait_scat(pb_v, b, b % 2)

        @pl.when(q < npq - 1)
        def _():
            load_panel(2 * q + 3, pb_v, sem_pb)
        return carry
    lax.fori_loop(0, npq, body, 0)
    plsc.subcore_barrier()

    for q in range(rpt // EB):
        r = row0 + q * EB
        pltpu.sync_copy(acc_sh.at[pl.ds(r, EB)], rows_a)
        pltpu.sync_copy(rows_a, out_hbm.at[c, pl.ds(r, EB)])


def _sc_deg(dst3, npad):
    nblk = dst3.shape[1]
    fn = pl.kernel(
        _sc_deg_body,
        out_type=jax.ShapeDtypeStruct((NW, npad), F32),
        mesh=_mesh(),
        compiler_params=pltpu.CompilerParams(needs_layout_passes=False),
        scratch_types=[
            pltpu.VMEM((nblk, EB), jnp.int32),
            pltpu.VMEM((npad,), F32),
        ],
    )
    return fn(dst3)


def _sc_agg(m, ei5, h):
    npad = m.shape[0]
    fn = pl.kernel(
        _sc_agg_body,
        out_type=jax.ShapeDtypeStruct((NC, npad, h), F32),
        mesh=_mesh(),
        compiler_params=pltpu.CompilerParams(needs_layout_passes=False),
        scratch_types=[
            pltpu.VMEM((PF, 2, EB), jnp.int32),
            pltpu.VMEM((PF, 2, EB), jnp.int32),
            pltpu.VMEM((EB, 128), F32),
            pltpu.VMEM((EB, 128), F32),
            pltpu.SemaphoreType.DMA,
            pltpu.SemaphoreType.DMA,
            pltpu.SemaphoreType.DMA,
            pltpu.SemaphoreType.DMA,
            pltpu.SemaphoreType.DMA,
            pltpu.SemaphoreType.DMA,
            pltpu.VMEM_SHARED((npad, 128), F32),
        ],
    )
    return fn(m, ei5)


def _tc0_body(degp_ref, x_ref, w_ref, dinv_ref, m_ref):
    deg = jnp.sum(degp_ref[...], axis=0) + 1.0
    dinv = lax.rsqrt(deg)[:, None]
    dinv_ref[...] = dinv
    t = jnp.dot(x_ref[...], w_ref[...], precision=HIGH,
                preferred_element_type=F32)
    m_ref[...] = t * dinv


def _tc_layer_body(p_ref, m_ref, dinv_ref, b_ref, w_ref, out_ref):
    dinv = dinv_ref[...]
    h = jnp.maximum((p_ref[0] + p_ref[1] + m_ref[...]) * dinv + b_ref[...],
                    0.0)
    t = jnp.dot(h, w_ref[...], precision=HIGH, preferred_element_type=F32)
    out_ref[...] = t * dinv


def _tc_final_body(p_ref, m_ref, dinv_ref, b_ref, batch_ref, wl_ref, bl_ref,
                   out_ref, pool_s, cnt_s):
    i = pl.program_id(0)

    @pl.when(i == 0)
    def _init():
        pool_s[...] = jnp.zeros_like(pool_s)
        cnt_s[...] = jnp.zeros_like(cnt_s)

    dinv = dinv_ref[...]
    h3 = (p_ref[0] + p_ref[1] + m_ref[...]) * dinv + b_ref[...]
    g = lax.broadcasted_iota(jnp.int32, (1, 128), 1)
    oh = (batch_ref[...] == g).astype(F32)
    pool_s[...] += lax.dot_general(oh, h3, (((0,), (0,)), ((), ())),
                                   precision=HIGH, preferred_element_type=F32)
    cnt_s[...] += jnp.sum(oh, axis=0)[None, :]

    @pl.when(i == pl.num_programs(0) - 1)
    def _fin():
        cnt = jnp.maximum(cnt_s[...], 1.0)
        pooled = pool_s[...][:64] / cnt[0, :64][:, None]
        out_ref[...] = jnp.dot(pooled, wl_ref[...], precision=HIGH,
                               preferred_element_type=F32) + bl_ref[...]


def _tc0(degp, xp, w1):
    npad, f = xp.shape
    h = w1.shape[1]
    return pl.pallas_call(
        _tc0_body,
        grid=(npad // RB,),
        in_specs=[
            pl.BlockSpec((NW, RB), lambda i: (0, i)),
            pl.BlockSpec((RB, f), lambda i: (i, 0)),
            pl.BlockSpec((f, h), lambda i: (0, 0)),
        ],
        out_specs=[
            pl.BlockSpec((RB, 1), lambda i: (i, 0)),
            pl.BlockSpec((RB, h), lambda i: (i, 0)),
        ],
        out_shape=[
            jax.ShapeDtypeStruct((npad, 1), F32),
            jax.ShapeDtypeStruct((npad, h), F32),
        ],
    )(degp, xp, w1)


def _tc_layer(p, m, dinv, b, w):
    npad, h = m.shape
    return pl.pallas_call(
        _tc_layer_body,
        grid=(npad // RB,),
        in_specs=[
            pl.BlockSpec((NC, RB, h), lambda i: (0, i, 0)),
            pl.BlockSpec((RB, h), lambda i: (i, 0)),
            pl.BlockSpec((RB, 1), lambda i: (i, 0)),
            pl.BlockSpec((1, h), lambda i: (0, 0)),
            pl.BlockSpec((h, h), lambda i: (0, 0)),
        ],
        out_specs=pl.BlockSpec((RB, h), lambda i: (i, 0)),
        out_shape=jax.ShapeDtypeStruct((npad, h), F32),
    )(p, m, dinv, b, w)


def _tc_final(p, m, dinv, b, batch2d, wlp, blp):
    npad, h = m.shape
    return pl.pallas_call(
        _tc_final_body,
        grid=(npad // RB,),
        in_specs=[
            pl.BlockSpec((NC, RB, h), lambda i: (0, i, 0)),
            pl.BlockSpec((RB, h), lambda i: (i, 0)),
            pl.BlockSpec((RB, 1), lambda i: (i, 0)),
            pl.BlockSpec((1, h), lambda i: (0, 0)),
            pl.BlockSpec((RB, 1), lambda i: (i, 0)),
            pl.BlockSpec((h, 128), lambda i: (0, 0)),
            pl.BlockSpec((1, 128), lambda i: (0, 0)),
        ],
        out_specs=pl.BlockSpec((64, 128), lambda i: (0, 0)),
        out_shape=jax.ShapeDtypeStruct((64, 128), F32),
        scratch_shapes=[
            pltpu.VMEM((128, 128), F32),
            pltpu.VMEM((1, 128), F32),
        ],
    )(p, m, dinv, b, batch2d, wlp, blp)


def kernel(x, edge_index, batch, W1, b1, W2, b2, W3, b3, Wl, bl):
    n, f = x.shape
    h = W1.shape[1]
    c = Wl.shape[1]
    e = edge_index.shape[1]
    npad = ((n + RB - 1) // RB) * RB

    xp = jnp.pad(x, ((0, npad - n), (0, 0)))
    batch2d = jnp.pad(batch, (0, npad - n), constant_values=127)[:, None]
    b1r = b1[None, :]
    b2r = b2[None, :]
    b3r = b3[None, :]
    wlp = jnp.pad(Wl, ((0, 0), (0, 128 - c)))
    blp = jnp.pad(bl, (0, 128 - c))[None, :]

    chunk = NW * PF * EB * 2
    ep = ((e + chunk - 1) // chunk) * chunk
    pad = ep - e
    pidx = jnp.arange(pad, dtype=edge_index.dtype)
    srcp = jnp.concatenate([edge_index[0], pidx % n])
    dstp = jnp.concatenate([edge_index[1], n + pidx % (npad - n)])
    ei5 = jnp.concatenate(
        [srcp.reshape(NW, -1, PF, 1, EB), dstp.reshape(NW, -1, PF, 1, EB)],
        axis=3)
    dst3 = dstp.reshape(NW, -1, EB)

    degp = _sc_deg(dst3, npad)
    dinv, m1 = _tc0(degp, xp, W1)
    p1 = _sc_agg(m1, ei5, h)
    m2 = _tc_layer(p1, m1, dinv, b1r, W2)
    p2 = _sc_agg(m2, ei5, h)
    m3 = _tc_layer(p2, m2, dinv, b2r, W3)
    p3 = _sc_agg(m3, ei5, h)
    out = _tc_final(p3, m3, dinv, b3r, batch2d, wlp, blp)
    return out[:, :c]


# speedup vs baseline: 1.0109x; 1.0109x over previous
import functools

import jax
import jax.numpy as jnp
from jax import lax
from jax.experimental import pallas as pl
from jax.experimental.pallas import tpu as pltpu
from jax.experimental.pallas import tpu_sc as plsc

NC = 2
NS = 16
NW = NC * NS
LANES = 16
EB = 128
PF = 4
RB = 1280
F32 = jnp.float32
HIGH = lax.Precision.HIGHEST


def _mesh():
    return plsc.VectorSubcoreMesh(
        core_axis_name="c", subcore_axis_name="s", num_cores=NC, num_subcores=NS
    )


def _sc_deg_body(dst3_hbm, out_hbm, idx_v, deg_v):
    c = lax.axis_index("c")
    s = lax.axis_index("s")
    wid = c * NS + s
    npad = deg_v.shape[0]
    nblk = dst3_hbm.shape[1]

    zeros16 = jnp.zeros((LANES,), F32)
    def zero_body(i, carry):
        deg_v[pl.ds(i * LANES, LANES)] = zeros16
        return carry
    lax.fori_loop(0, npad // LANES, zero_body, 0)

    pltpu.sync_copy(dst3_hbm.at[wid], idx_v)
    ones16 = jnp.ones((LANES,), F32)

    def body(j, carry):
        for k in range(EB // LANES):
            d = idx_v[j, pl.ds(k * LANES, LANES)]
            plsc.addupdate_scatter(deg_v, [d], ones16)
        return carry
    lax.fori_loop(0, nblk, body, 0)

    pltpu.sync_copy(deg_v, out_hbm.at[wid])


def _sc_agg_body(m_hbm, ei5_hbm, out_hbm, pa_v, pb_v, rows_a, rows_b,
                 sem_pa, sem_pb, sem_a0, sem_b0, acc_sh):
    c = lax.axis_index("c")
    s = lax.axis_index("s")
    wid = c * NS + s
    npad = m_hbm.shape[0]
    npanel = ei5_hbm.shape[1]
    rpt = npad // NS
    row0 = s * rpt

    zeros16 = jnp.zeros((LANES,), F32)
    def zero_body(i, carry):
        for k in range(128 // LANES):
            rows_a[i, pl.ds(k * LANES, LANES)] = zeros16
        return carry
    lax.fori_loop(0, EB, zero_body, 0)
    for q in range(rpt // EB):
        pltpu.sync_copy(rows_a, acc_sh.at[pl.ds(row0 + q * EB, EB)])
    plsc.subcore_barrier()

    slots = (rows_a, rows_b)
    sems = (sem_a0, sem_b0)

    def start_gather(panel, b, sl):
        pltpu.async_copy(m_hbm.at[panel.at[b, 0]], slots[sl], sems[sl])

    def wait_scat(panel, b, sl):
        pltpu.make_async_copy(m_hbm.at[panel.at[b, 0]], slots[sl],
                              sems[sl]).wait()
        pltpu.sync_copy(slots[sl], acc_sh.at[panel.at[b, 1]], add=True)

    def load_panel(p_idx, panel, sem):
        pltpu.async_copy(ei5_hbm.at[wid, p_idx], panel, sem)

    def wait_panel(panel, sem):
        pltpu.make_async_copy(ei5_hbm.at[wid, 0], panel, sem).wait()

    pltpu.sync_copy(ei5_hbm.at[wid, 0], pa_v)
    start_gather(pa_v, 0, 0)
    load_panel(1, pb_v, sem_pb)

    npq = npanel // 2

    def body(q, carry):
        for b in range(PF):
            if b < PF - 1:
                start_gather(pa_v, b + 1, (b + 1) % 2)
            else:
                wait_panel(pb_v, sem_pb)
                start_gather(pb_v, 0, 0)
            wait_scat(pa_v, b, b % 2)

        @pl.when(q < npq - 1)
        def _():
            load_panel(2 * q + 2, pa_v, sem_pa)

        for b in range(PF):
            if b < PF - 1:
                start_gather(pb_v, b + 1, (b + 1) % 2)
            else:
                @pl.when(q < npq - 1)
                def _():
                    wait_panel(pa_v, sem_pa)
                    start_gather(pa_v, 0, 0)
            wait_scat(pb_v, b, b % 2)

        @pl.when(q < npq - 1)
        def _():
            load_panel(2 * q + 3, pb_v, sem_pb)
        return carry
    lax.fori_loop(0, npq, body, 0)
    plsc.subcore_barrier()

    for q in range(rpt // EB):
        r = row0 + q * EB
        pltpu.sync_copy(acc_sh.at[pl.ds(r, EB)], rows_a)
        pltpu.sync_copy(rows_a, out_hbm.at[c, pl.ds(r, EB)])


def _sc_deg(dst3, npad):
    nblk = dst3.shape[1]
    fn = pl.kernel(
        _sc_deg_body,
        out_type=jax.ShapeDtypeStruct((NW, npad), F32),
        mesh=_mesh(),
        compiler_params=pltpu.CompilerParams(needs_layout_passes=False),
        scratch_types=[
            pltpu.VMEM((nblk, EB), jnp.int32),
            pltpu.VMEM((npad,), F32),
        ],
    )
    return fn(dst3)


def _sc_agg(m, ei5, h):
    npad = m.shape[0]
    fn = pl.kernel(
        _sc_agg_body,
        out_type=jax.ShapeDtypeStruct((NC, npad, h), F32),
        mesh=_mesh(),
        compiler_params=pltpu.CompilerParams(needs_layout_passes=False),
        scratch_types=[
            pltpu.VMEM((PF, 2, EB), jnp.int32),
            pltpu.VMEM((PF, 2, EB), jnp.int32),
            pltpu.VMEM((EB, 128), F32),
            pltpu.VMEM((EB, 128), F32),
            pltpu.SemaphoreType.DMA,
            pltpu.SemaphoreType.DMA,
            pltpu.SemaphoreType.DMA,
            pltpu.SemaphoreType.DMA,
            pltpu.VMEM_SHARED((npad, 128), F32),
        ],
    )
    return fn(m, ei5)


def _tc_mm_body(x_ref, w_ref, t_ref):
    t_ref[...] = jnp.dot(x_ref[...], w_ref[...], precision=HIGH,
                         preferred_element_type=F32)


def _tc0_body(degp_ref, t_ref, dinv_ref, m_ref):
    deg = jnp.sum(degp_ref[...], axis=0) + 1.0
    dinv = lax.rsqrt(deg)[:, None]
    dinv_ref[...] = dinv
    m_ref[...] = t_ref[...] * dinv


def _tc_layer_body(p_ref, m_ref, dinv_ref, b_ref, w_ref, out_ref):
    dinv = dinv_ref[...]
    h = jnp.maximum((p_ref[0] + p_ref[1] + m_ref[...]) * dinv + b_ref[...],
                    0.0)
    t = jnp.dot(h, w_ref[...], precision=HIGH, preferred_element_type=F32)
    out_ref[...] = t * dinv


def _tc_final_body(p_ref, m_ref, dinv_ref, b_ref, batch_ref, wl_ref, bl_ref,
                   out_ref, pool_s, cnt_s):
    i = pl.program_id(0)

    @pl.when(i == 0)
    def _init():
        pool_s[...] = jnp.zeros_like(pool_s)
        cnt_s[...] = jnp.zeros_like(cnt_s)

    dinv = dinv_ref[...]
    h3 = (p_ref[0] + p_ref[1] + m_ref[...]) * dinv + b_ref[...]
    g = lax.broadcasted_iota(jnp.int32, (1, 128), 1)
    oh = (batch_ref[...] == g).astype(F32)
    pool_s[...] += lax.dot_general(oh, h3, (((0,), (0,)), ((), ())),
                                   precision=HIGH, preferred_element_type=F32)
    cnt_s[...] += jnp.sum(oh, axis=0)[None, :]

    @pl.when(i == pl.num_programs(0) - 1)
    def _fin():
        cnt = jnp.maximum(cnt_s[...], 1.0)
        pooled = pool_s[...][:64] / cnt[0, :64][:, None]
        out_ref[...] = jnp.dot(pooled, wl_ref[...], precision=HIGH,
                               preferred_element_type=F32) + bl_ref[...]


def _tc_mm(xp, w1):
    npad, f = xp.shape
    h = w1.shape[1]
    return pl.pallas_call(
        _tc_mm_body,
        grid=(npad // RB,),
        in_specs=[
            pl.BlockSpec((RB, f), lambda i: (i, 0)),
            pl.BlockSpec((f, h), lambda i: (0, 0)),
        ],
        out_specs=pl.BlockSpec((RB, h), lambda i: (i, 0)),
        out_shape=jax.ShapeDtypeStruct((npad, h), F32),
    )(xp, w1)


def _tc0(degp, t1):
    npad, h = t1.shape
    return pl.pallas_call(
        _tc0_body,
        grid=(npad // RB,),
        in_specs=[
            pl.BlockSpec((NW, RB), lambda i: (0, i)),
            pl.BlockSpec((RB, h), lambda i: (i, 0)),
        ],
        out_specs=[
            pl.BlockSpec((RB, 1), lambda i: (i, 0)),
            pl.BlockSpec((RB, h), lambda i: (i, 0)),
        ],
        out_shape=[
            jax.ShapeDtypeStruct((npad, 1), F32),
            jax.ShapeDtypeStruct((npad, h), F32),
        ],
    )(degp, t1)


def _tc_layer(p, m, dinv, b, w):
    npad, h = m.shape
    return pl.pallas_call(
        _tc_layer_body,
        grid=(npad // RB,),
        in_specs=[
            pl.BlockSpec((NC, RB, h), lambda i: (0, i, 0)),
            pl.BlockSpec((RB, h), lambda i: (i, 0)),
            pl.BlockSpec((RB, 1), lambda i: (i, 0)),
            pl.BlockSpec((1, h), lambda i: (0, 0)),
            pl.BlockSpec((h, h), lambda i: (0, 0)),
        ],
        out_specs=pl.BlockSpec((RB, h), lambda i: (i, 0)),
        out_shape=jax.ShapeDtypeStruct((npad, h), F32),
    )(p, m, dinv, b, w)


def _tc_final(p, m, dinv, b, batch2d, wlp, blp):
    npad, h = m.shape
    return pl.pallas_call(
        _tc_final_body,
        grid=(npad // RB,),
        in_specs=[
            pl.BlockSpec((NC, RB, h), lambda i: (0, i, 0)),
            pl.BlockSpec((RB, h), lambda i: (i, 0)),
            pl.BlockSpec((RB, 1), lambda i: (i, 0)),
            pl.BlockSpec((1, h), lambda i: (0, 0)),
            pl.BlockSpec((RB, 1), lambda i: (i, 0)),
            pl.BlockSpec((h, 128), lambda i: (0, 0)),
            pl.BlockSpec((1, 128), lambda i: (0, 0)),
        ],
        out_specs=pl.BlockSpec((64, 128), lambda i: (0, 0)),
        out_shape=jax.ShapeDtypeStruct((64, 128), F32),
        scratch_shapes=[
            pltpu.VMEM((128, 128), F32),
            pltpu.VMEM((1, 128), F32),
        ],
    )(p, m, dinv, b, batch2d, wlp, blp)


def kernel(x, edge_index, batch, W1, b1, W2, b2, W3, b3, Wl, bl):
    n, f = x.shape
    h = W1.shape[1]
    c = Wl.shape[1]
    e = edge_index.shape[1]
    npad = ((n + RB - 1) // RB) * RB

    xp = jnp.pad(x, ((0, npad - n), (0, 0)))
    batch2d = jnp.pad(batch, (0, npad - n), constant_values=127)[:, None]
    b1r = b1[None, :]
    b2r = b2[None, :]
    b3r = b3[None, :]
    wlp = jnp.pad(Wl, ((0, 0), (0, 128 - c)))
    blp = jnp.pad(bl, (0, 128 - c))[None, :]

    chunk = NW * PF * EB * 2
    ep = ((e + chunk - 1) // chunk) * chunk
    pad = ep - e
    pidx = jnp.arange(pad, dtype=edge_index.dtype)
    srcp = jnp.concatenate([edge_index[0], pidx % n])
    dstp = jnp.concatenate([edge_index[1], n + pidx % (npad - n)])
    ei5 = jnp.concatenate(
        [srcp.reshape(NW, -1, PF, 1, EB), dstp.reshape(NW, -1, PF, 1, EB)],
        axis=3)
    dst3 = dstp.reshape(NW, -1, EB)

    t1 = _tc_mm(xp, W1)
    degp = _sc_deg(dst3, npad)
    dinv, m1 = _tc0(degp, t1)
    p1 = _sc_agg(m1, ei5, h)
    m2 = _tc_layer(p1, m1, dinv, b1r, W2)
    p2 = _sc_agg(m2, ei5, h)
    m3 = _tc_layer(p2, m2, dinv, b2r, W3)
    p3 = _sc_agg(m3, ei5, h)
    out = _tc_final(p3, m3, dinv, b3r, batch2d, wlp, blp)
    return out[:, :c]


# speedup vs baseline: 1.1060x; 1.0940x over previous
import functools

import jax
import jax.numpy as jnp
from jax import lax
from jax.experimental import pallas as pl
from jax.experimental.pallas import tpu as pltpu
from jax.experimental.pallas import tpu_sc as plsc

NC = 2
NS = 16
NW = NC * NS
LANES = 16
EB = 128
PF = 4
RB = 1280
F32 = jnp.float32
HIGH = lax.Precision.HIGHEST


def _mesh():
    return plsc.VectorSubcoreMesh(
        core_axis_name="c", subcore_axis_name="s", num_cores=NC, num_subcores=NS
    )


def _sc_deg_body(dst3_hbm, out_hbm, idx_v, deg_v):
    c = lax.axis_index("c")
    s = lax.axis_index("s")
    wid = c * NS + s
    npad = deg_v.shape[0]
    nblk = dst3_hbm.shape[1]

    zeros16 = jnp.zeros((LANES,), F32)
    def zero_body(i, carry):
        deg_v[pl.ds(i * LANES, LANES)] = zeros16
        return carry
    lax.fori_loop(0, npad // LANES, zero_body, 0)

    pltpu.sync_copy(dst3_hbm.at[wid], idx_v)
    ones16 = jnp.ones((LANES,), F32)

    def body(j, carry):
        for k in range(EB // LANES):
            d = idx_v[j, pl.ds(k * LANES, LANES)]
            plsc.addupdate_scatter(deg_v, [d], ones16)
        return carry
    lax.fori_loop(0, nblk, body, 0)

    pltpu.sync_copy(deg_v, out_hbm.at[wid])


def _sc_agg_body(m_hbm, ei5_hbm, out_hbm, pa_v, pb_v, rows_a, rows_b,
                 sem_pa, sem_pb, sem_a0, sem_b0, acc_sh, *, gi=0, si=1):
    c = lax.axis_index("c")
    s = lax.axis_index("s")
    wid = c * NS + s
    npad = m_hbm.shape[0]
    width = m_hbm.shape[1]
    npanel = ei5_hbm.shape[1]
    rpt = npad // NS
    row0 = s * rpt

    zeros16 = jnp.zeros((LANES,), F32)
    def zero_body(i, carry):
        for k in range(width // LANES):
            rows_a[i, pl.ds(k * LANES, LANES)] = zeros16
        return carry
    lax.fori_loop(0, EB, zero_body, 0)
    for q in range(rpt // EB):
        pltpu.sync_copy(rows_a, acc_sh.at[pl.ds(row0 + q * EB, EB)])
    plsc.subcore_barrier()

    slots = (rows_a, rows_b)
    sems = (sem_a0, sem_b0)

    def start_gather(panel, b, sl):
        pltpu.async_copy(m_hbm.at[panel.at[b, gi]], slots[sl], sems[sl])

    def wait_scat(panel, b, sl):
        pltpu.make_async_copy(m_hbm.at[panel.at[b, gi]], slots[sl],
                              sems[sl]).wait()
        pltpu.sync_copy(slots[sl], acc_sh.at[panel.at[b, si]], add=True)

    def load_panel(p_idx, panel, sem):
        pltpu.async_copy(ei5_hbm.at[wid, p_idx], panel, sem)

    def wait_panel(panel, sem):
        pltpu.make_async_copy(ei5_hbm.at[wid, 0], panel, sem).wait()

    pltpu.sync_copy(ei5_hbm.at[wid, 0], pa_v)
    start_gather(pa_v, 0, 0)
    load_panel(1, pb_v, sem_pb)

    npq = npanel // 2

    def body(q, carry):
        for b in range(PF):
            if b < PF - 1:
                start_gather(pa_v, b + 1, (b + 1) % 2)
            else:
                wait_panel(pb_v, sem_pb)
                start_gather(pb_v, 0, 0)
            wait_scat(pa_v, b, b % 2)

        @pl.when(q < npq - 1)
        def _():
            load_panel(2 * q + 2, pa_v, sem_pa)

        for b in range(PF):
            if b < PF - 1:
                start_gather(pb_v, b + 1, (b + 1) % 2)
            else:
                @pl.when(q < npq - 1)
                def _():
                    wait_panel(pa_v, sem_pa)
                    start_gather(pa_v, 0, 0)
            wait_scat(pb_v, b, b % 2)

        @pl.when(q < npq - 1)
        def _():
            load_panel(2 * q + 3, pb_v, sem_pb)
        return carry
    lax.fori_loop(0, npq, body, 0)
    plsc.subcore_barrier()

    for q in range(rpt // EB):
        r = row0 + q * EB
        pltpu.sync_copy(acc_sh.at[pl.ds(r, EB)], rows_a)
        pltpu.sync_copy(rows_a, out_hbm.at[c, pl.ds(r, EB)])


def _sc_deg(dst3, npad):
    nblk = dst3.shape[1]
    fn = pl.kernel(
        _sc_deg_body,
        out_type=jax.ShapeDtypeStruct((NW, npad), F32),
        mesh=_mesh(),
        compiler_params=pltpu.CompilerParams(needs_layout_passes=False),
        scratch_types=[
            pltpu.VMEM((nblk, EB), jnp.int32),
            pltpu.VMEM((npad,), F32),
        ],
    )
    return fn(dst3)


def _sc_agg(m, ei5, swap=False):
    npad, width = m.shape
    fn = pl.kernel(
        functools.partial(_sc_agg_body, gi=1 if swap else 0,
                          si=0 if swap else 1),
        out_type=jax.ShapeDtypeStruct((NC, npad, width), F32),
        mesh=_mesh(),
        compiler_params=pltpu.CompilerParams(needs_layout_passes=False,
                                             use_tc_tiling_on_sc=False),
        scratch_types=[
            pltpu.VMEM((PF, 2, EB), jnp.int32),
            pltpu.VMEM((PF, 2, EB), jnp.int32),
            pltpu.VMEM((EB, width), F32),
            pltpu.VMEM((EB, width), F32),
            pltpu.SemaphoreType.DMA,
            pltpu.SemaphoreType.DMA,
            pltpu.SemaphoreType.DMA,
            pltpu.SemaphoreType.DMA,
            pltpu.VMEM_SHARED((npad, width), F32),
        ],
    )
    return fn(m, ei5)


def _tc_mm_body(x_ref, w_ref, t_ref):
    t_ref[...] = jnp.dot(x_ref[...], w_ref[...], precision=HIGH,
                         preferred_element_type=F32)


def _tc0_body(degp_ref, t_ref, batch_ref, dinv_ref, m_ref, r_ref):
    deg = jnp.sum(degp_ref[...], axis=0) + 1.0
    dinv = lax.rsqrt(deg)[:, None]
    dinv_ref[...] = dinv
    m_ref[...] = t_ref[...] * dinv
    g = lax.broadcasted_iota(jnp.int32, (1, 64), 1)
    r_ref[...] = (batch_ref[...] == g).astype(F32) * dinv


def _tc_layer_body(p_ref, m_ref, dinv_ref, b_ref, w_ref, out_ref):
    dinv = dinv_ref[...]
    h = jnp.maximum((p_ref[0] + p_ref[1] + m_ref[...]) * dinv + b_ref[...],
                    0.0)
    t = jnp.dot(h, w_ref[...], precision=HIGH, preferred_element_type=F32)
    out_ref[...] = t * dinv


def _tc_final_body(cp_ref, m_ref, dinv_ref, b_ref, batch_ref, wl_ref, bl_ref,
                   out_ref, pool_s, agg_s, cnt_s):
    i = pl.program_id(0)

    @pl.when(i == 0)
    def _init():
        pool_s[...] = jnp.zeros_like(pool_s)
        agg_s[...] = jnp.zeros_like(agg_s)
        cnt_s[...] = jnp.zeros_like(cnt_s)

    dinv = dinv_ref[...]
    m3 = m_ref[...]
    h3self = m3 * dinv
    g = lax.broadcasted_iota(jnp.int32, (1, 128), 1)
    oh = (batch_ref[...] == g).astype(F32)
    pool_s[...] += lax.dot_general(oh, h3self, (((0,), (0,)), ((), ())),
                                   precision=HIGH, preferred_element_type=F32)
    cb = cp_ref[0] + cp_ref[1]
    agg_s[...] += lax.dot_general(cb, m3, (((0,), (0,)), ((), ())),
                                  precision=HIGH, preferred_element_type=F32)
    cnt_s[...] += jnp.sum(oh, axis=0)[None, :]

    @pl.when(i == pl.num_programs(0) - 1)
    def _fin():
        cnt = jnp.maximum(cnt_s[...], 1.0)
        c64 = cnt[0, :64][:, None]
        total = pool_s[...][:64] + agg_s[...] + c64 * b_ref[...]
        pooled = total / c64
        out_ref[...] = jnp.dot(pooled, wl_ref[...], precision=HIGH,
                               preferred_element_type=F32) + bl_ref[...]


def _tc_mm(xp, w1):
    npad, f = xp.shape
    h = w1.shape[1]
    return pl.pallas_call(
        _tc_mm_body,
        grid=(npad // RB,),
        in_specs=[
            pl.BlockSpec((RB, f), lambda i: (i, 0)),
            pl.BlockSpec((f, h), lambda i: (0, 0)),
        ],
        out_specs=pl.BlockSpec((RB, h), lambda i: (i, 0)),
        out_shape=jax.ShapeDtypeStruct((npad, h), F32),
    )(xp, w1)


def _tc0(degp, t1, batch2d):
    npad, h = t1.shape
    return pl.pallas_call(
        _tc0_body,
        grid=(npad // RB,),
        in_specs=[
            pl.BlockSpec((NW, RB), lambda i: (0, i)),
            pl.BlockSpec((RB, h), lambda i: (i, 0)),
            pl.BlockSpec((RB, 1), lambda i: (i, 0)),
        ],
        out_specs=[
            pl.BlockSpec((RB, 1), lambda i: (i, 0)),
            pl.BlockSpec((RB, h), lambda i: (i, 0)),
            pl.BlockSpec((RB, 64), lambda i: (i, 0)),
        ],
        out_shape=[
            jax.ShapeDtypeStruct((npad, 1), F32),
            jax.ShapeDtypeStruct((npad, h), F32),
            jax.ShapeDtypeStruct((npad, 64), F32),
        ],
    )(degp, t1, batch2d)


def _tc_layer(p, m, dinv, b, w):
    npad, h = m.shape
    return pl.pallas_call(
        _tc_layer_body,
        grid=(npad // RB,),
        in_specs=[
            pl.BlockSpec((NC, RB, h), lambda i: (0, i, 0)),
            pl.BlockSpec((RB, h), lambda i: (i, 0)),
            pl.BlockSpec((RB, 1), lambda i: (i, 0)),
            pl.BlockSpec((1, h), lambda i: (0, 0)),
            pl.BlockSpec((h, h), lambda i: (0, 0)),
        ],
        out_specs=pl.BlockSpec((RB, h), lambda i: (i, 0)),
        out_shape=jax.ShapeDtypeStruct((npad, h), F32),
    )(p, m, dinv, b, w)


def _tc_final(cp, m, dinv, b, batch2d, wlp, blp):
    npad, h = m.shape
    return pl.pallas_call(
        _tc_final_body,
        grid=(npad // RB,),
        in_specs=[
            pl.BlockSpec((NC, RB, 64), lambda i: (0, i, 0)),
            pl.BlockSpec((RB, h), lambda i: (i, 0)),
            pl.BlockSpec((RB, 1), lambda i: (i, 0)),
            pl.BlockSpec((1, h), lambda i: (0, 0)),
            pl.BlockSpec((RB, 1), lambda i: (i, 0)),
            pl.BlockSpec((h, 128), lambda i: (0, 0)),
            pl.BlockSpec((1, 128), lambda i: (0, 0)),
        ],
        out_specs=pl.BlockSpec((64, 128), lambda i: (0, 0)),
        out_shape=jax.ShapeDtypeStruct((64, 128), F32),
        scratch_shapes=[
            pltpu.VMEM((128, 128), F32),
            pltpu.VMEM((64, 128), F32),
            pltpu.VMEM((1, 128), F32),
        ],
    )(cp, m, dinv, b, batch2d, wlp, blp)


def kernel(x, edge_index, batch, W1, b1, W2, b2, W3, b3, Wl, bl):
    n, f = x.shape
    h = W1.shape[1]
    c = Wl.shape[1]
    e = edge_index.shape[1]
    npad = ((n + RB - 1) // RB) * RB

    xp = jnp.pad(x, ((0, npad - n), (0, 0)))
    batch2d = jnp.pad(batch, (0, npad - n), constant_values=127)[:, None]
    b1r = b1[None, :]
    b2r = b2[None, :]
    b3r = b3[None, :]
    wlp = jnp.pad(Wl, ((0, 0), (0, 128 - c)))
    blp = jnp.pad(bl, (0, 128 - c))[None, :]

    chunk = NW * PF * EB * 2
    ep = ((e + chunk - 1) // chunk) * chunk
    pad = ep - e
    pidx = jnp.arange(pad, dtype=edge_index.dtype)
    srcp = jnp.concatenate([edge_index[0], pidx % n])
    dstp = jnp.concatenate([edge_index[1], n + pidx % (npad - n)])
    ei5 = jnp.concatenate(
        [srcp.reshape(NW, -1, PF, 1, EB), dstp.reshape(NW, -1, PF, 1, EB)],
        axis=3)
    dst3 = dstp.reshape(NW, -1, EB)

    degp = _sc_deg(dst3, npad)
    t1 = _tc_mm(xp, W1)
    dinv, m1, r = _tc0(degp, t1, batch2d)
    p1 = _sc_agg(m1, ei5)
    m2 = _tc_layer(p1, m1, dinv, b1r, W2)
    p2 = _sc_agg(m2, ei5)
    m3 = _tc_layer(p2, m2, dinv, b2r, W3)
    cp = _sc_agg(r, ei5, swap=True)
    out = _tc_final(cp, m3, dinv, b3r, batch2d, wlp, blp)
    return out[:, :c]


# speedup vs baseline: 1.1344x; 1.0257x over previous
import functools

import jax
import jax.numpy as jnp
from jax import lax
from jax.experimental import pallas as pl
from jax.experimental.pallas import tpu as pltpu
from jax.experimental.pallas import tpu_sc as plsc

NC = 2
NS = 16
NW = NC * NS
LANES = 16
EB = 128
PF = 4
RB = 1280
F32 = jnp.float32
HIGH = lax.Precision.HIGHEST


def _mesh():
    return plsc.VectorSubcoreMesh(
        core_axis_name="c", subcore_axis_name="s", num_cores=NC, num_subcores=NS
    )


def _sc_deg_body(dst3_hbm, out_hbm, idx_v, deg_v):
    c = lax.axis_index("c")
    s = lax.axis_index("s")
    wid = c * NS + s
    npad = deg_v.shape[0]
    nblk = dst3_hbm.shape[1]

    zeros16 = jnp.zeros((LANES,), F32)
    def zero_body(i, carry):
        deg_v[pl.ds(i * LANES, LANES)] = zeros16
        return carry
    lax.fori_loop(0, npad // LANES, zero_body, 0)

    pltpu.sync_copy(dst3_hbm.at[wid], idx_v)
    ones16 = jnp.ones((LANES,), F32)

    def body(j, carry):
        for k in range(EB // LANES):
            d = idx_v[j, pl.ds(k * LANES, LANES)]
            plsc.addupdate_scatter(deg_v, [d], ones16)
        return carry
    lax.fori_loop(0, nblk, body, 0)

    pltpu.sync_copy(deg_v, out_hbm.at[wid])


def _sc_agg_body(m_hbm, ei5_hbm, out_hbm, pa_v, pb_v, rows_a, rows_b,
                 sem_pa, sem_pb, sem_a0, sem_b0, acc_sh, *, gi=0, si=1):
    c = lax.axis_index("c")
    s = lax.axis_index("s")
    wid = c * NS + s
    npad = m_hbm.shape[0]
    width = m_hbm.shape[1]
    npanel = ei5_hbm.shape[1]
    rpt = npad // NS
    row0 = s * rpt

    zeros16 = jnp.zeros((LANES,), F32)
    def zero_body(i, carry):
        for k in range(width // LANES):
            rows_a[i, pl.ds(k * LANES, LANES)] = zeros16
        return carry
    lax.fori_loop(0, EB, zero_body, 0)
    for q in range(rpt // EB):
        pltpu.sync_copy(rows_a, acc_sh.at[pl.ds(row0 + q * EB, EB)])
    plsc.subcore_barrier()

    slots = (rows_a, rows_b)
    sems = (sem_a0, sem_b0)

    def start_gather(panel, b, sl):
        pltpu.async_copy(m_hbm.at[panel.at[b, gi]], slots[sl], sems[sl])

    def wait_scat(panel, b, sl):
        pltpu.make_async_copy(m_hbm.at[panel.at[b, gi]], slots[sl],
                              sems[sl]).wait()
        pltpu.sync_copy(slots[sl], acc_sh.at[panel.at[b, si]], add=True)

    def load_panel(p_idx, panel, sem):
        pltpu.async_copy(ei5_hbm.at[wid, p_idx], panel, sem)

    def wait_panel(panel, sem):
        pltpu.make_async_copy(ei5_hbm.at[wid, 0], panel, sem).wait()

    pltpu.sync_copy(ei5_hbm.at[wid, 0], pa_v)
    start_gather(pa_v, 0, 0)
    load_panel(1, pb_v, sem_pb)

    npq = npanel // 2

    def body(q, carry):
        for b in range(PF):
            if b < PF - 1:
                start_gather(pa_v, b + 1, (b + 1) % 2)
            else:
                wait_panel(pb_v, sem_pb)
                start_gather(pb_v, 0, 0)
            wait_scat(pa_v, b, b % 2)

        @pl.when(q < npq - 1)
        def _():
            load_panel(2 * q + 2, pa_v, sem_pa)

        for b in range(PF):
            if b < PF - 1:
                start_gather(pb_v, b + 1, (b + 1) % 2)
            else:
                @pl.when(q < npq - 1)
                def _():
                    wait_panel(pa_v, sem_pa)
                    start_gather(pa_v, 0, 0)
            wait_scat(pb_v, b, b % 2)

        @pl.when(q < npq - 1)
        def _():
            load_panel(2 * q + 3, pb_v, sem_pb)
        return carry
    lax.fori_loop(0, npq, body, 0)
    plsc.subcore_barrier()

    pltpu.sync_copy(acc_sh.at[pl.ds(row0, rpt)], out_hbm.at[c, pl.ds(row0, rpt)])


def _sc_deg(dst3, npad):
    nblk = dst3.shape[1]
    fn = pl.kernel(
        _sc_deg_body,
        out_type=jax.ShapeDtypeStruct((NW, npad), F32),
        mesh=_mesh(),
        compiler_params=pltpu.CompilerParams(needs_layout_passes=False),
        scratch_types=[
            pltpu.VMEM((nblk, EB), jnp.int32),
            pltpu.VMEM((npad,), F32),
        ],
    )
    return fn(dst3)


def _sc_agg(m, ei5, swap=False):
    npad, width = m.shape
    fn = pl.kernel(
        functools.partial(_sc_agg_body, gi=1 if swap else 0,
                          si=0 if swap else 1),
        out_type=jax.ShapeDtypeStruct((NC, npad, width), F32),
        mesh=_mesh(),
        compiler_params=pltpu.CompilerParams(needs_layout_passes=False,
                                             use_tc_tiling_on_sc=False),
        scratch_types=[
            pltpu.VMEM((PF, 2, EB), jnp.int32),
            pltpu.VMEM((PF, 2, EB), jnp.int32),
            pltpu.VMEM((EB, width), F32),
            pltpu.VMEM((EB, width), F32),
            pltpu.SemaphoreType.DMA,
            pltpu.SemaphoreType.DMA,
            pltpu.SemaphoreType.DMA,
            pltpu.SemaphoreType.DMA,
            pltpu.VMEM_SHARED((npad, width), F32),
        ],
    )
    return fn(m, ei5)


def _tc0_body(degp_ref, x_ref, w_ref, batch_ref, dinv_ref, m_ref, r_ref):
    deg = jnp.sum(degp_ref[...], axis=0) + 1.0
    dinv = lax.rsqrt(deg)[:, None]
    dinv_ref[...] = dinv
    t = jnp.dot(x_ref[...], w_ref[...], precision=HIGH,
                preferred_element_type=F32)
    m_ref[...] = t * dinv
    g = lax.broadcasted_iota(jnp.int32, (1, 64), 1)
    r_ref[...] = (batch_ref[...] == g).astype(F32) * dinv


def _tc_layer_body(p_ref, m_ref, dinv_ref, b_ref, w_ref, out_ref):
    dinv = dinv_ref[...]
    h = jnp.maximum((p_ref[0] + p_ref[1] + m_ref[...]) * dinv + b_ref[...],
                    0.0)
    t = jnp.dot(h, w_ref[...], precision=HIGH, preferred_element_type=F32)
    out_ref[...] = t * dinv


def _tc_final_body(cp_ref, m_ref, dinv_ref, b_ref, batch_ref, wl_ref, bl_ref,
                   out_ref, pool_s, agg_s, cnt_s):
    i = pl.program_id(0)

    @pl.when(i == 0)
    def _init():
        pool_s[...] = jnp.zeros_like(pool_s)
        agg_s[...] = jnp.zeros_like(agg_s)
        cnt_s[...] = jnp.zeros_like(cnt_s)

    dinv = dinv_ref[...]
    m3 = m_ref[...]
    h3self = m3 * dinv
    g = lax.broadcasted_iota(jnp.int32, (1, 128), 1)
    oh = (batch_ref[...] == g).astype(F32)
    pool_s[...] += lax.dot_general(oh, h3self, (((0,), (0,)), ((), ())),
                                   precision=HIGH, preferred_element_type=F32)
    cb = cp_ref[0] + cp_ref[1]
    agg_s[...] += lax.dot_general(cb, m3, (((0,), (0,)), ((), ())),
                                  precision=HIGH, preferred_element_type=F32)
    cnt_s[...] += jnp.sum(oh, axis=0)[None, :]

    @pl.when(i == pl.num_programs(0) - 1)
    def _fin():
        cnt = jnp.maximum(cnt_s[...], 1.0)
        c64 = cnt[0, :64][:, None]
        total = pool_s[...][:64] + agg_s[...] + c64 * b_ref[...]
        pooled = total / c64
        out_ref[...] = jnp.dot(pooled, wl_ref[...], precision=HIGH,
                               preferred_element_type=F32) + bl_ref[...]


def _tc0(degp, xp, w1, batch2d):
    npad, f = xp.shape
    h = w1.shape[1]
    return pl.pallas_call(
        _tc0_body,
        grid=(npad // RB,),
        in_specs=[
            pl.BlockSpec((NW, RB), lambda i: (0, i)),
            pl.BlockSpec((RB, f), lambda i: (i, 0)),
            pl.BlockSpec((f, h), lambda i: (0, 0)),
            pl.BlockSpec((RB, 1), lambda i: (i, 0)),
        ],
        out_specs=[
            pl.BlockSpec((RB, 1), lambda i: (i, 0)),
            pl.BlockSpec((RB, h), lambda i: (i, 0)),
            pl.BlockSpec((RB, 64), lambda i: (i, 0)),
        ],
        out_shape=[
            jax.ShapeDtypeStruct((npad, 1), F32),
            jax.ShapeDtypeStruct((npad, h), F32),
            jax.ShapeDtypeStruct((npad, 64), F32),
        ],
    )(degp, xp, w1, batch2d)


def _tc_layer(p, m, dinv, b, w):
    npad, h = m.shape
    return pl.pallas_call(
        _tc_layer_body,
        grid=(npad // RB,),
        in_specs=[
            pl.BlockSpec((NC, RB, h), lambda i: (0, i, 0)),
            pl.BlockSpec((RB, h), lambda i: (i, 0)),
            pl.BlockSpec((RB, 1), lambda i: (i, 0)),
            pl.BlockSpec((1, h), lambda i: (0, 0)),
            pl.BlockSpec((h, h), lambda i: (0, 0)),
        ],
        out_specs=pl.BlockSpec((RB, h), lambda i: (i, 0)),
        out_shape=jax.ShapeDtypeStruct((npad, h), F32),
    )(p, m, dinv, b, w)


def _tc_final(cp, m, dinv, b, batch2d, wlp, blp):
    npad, h = m.shape
    return pl.pallas_call(
        _tc_final_body,
        grid=(npad // RB,),
        in_specs=[
            pl.BlockSpec((NC, RB, 64), lambda i: (0, i, 0)),
            pl.BlockSpec((RB, h), lambda i: (i, 0)),
            pl.BlockSpec((RB, 1), lambda i: (i, 0)),
            pl.BlockSpec((1, h), lambda i: (0, 0)),
            pl.BlockSpec((RB, 1), lambda i: (i, 0)),
            pl.BlockSpec((h, 128), lambda i: (0, 0)),
            pl.BlockSpec((1, 128), lambda i: (0, 0)),
        ],
        out_specs=pl.BlockSpec((64, 128), lambda i: (0, 0)),
        out_shape=jax.ShapeDtypeStruct((64, 128), F32),
        scratch_shapes=[
            pltpu.VMEM((128, 128), F32),
            pltpu.VMEM((64, 128), F32),
            pltpu.VMEM((1, 128), F32),
        ],
    )(cp, m, dinv, b, batch2d, wlp, blp)


def kernel(x, edge_index, batch, W1, b1, W2, b2, W3, b3, Wl, bl):
    n, f = x.shape
    h = W1.shape[1]
    c = Wl.shape[1]
    e = edge_index.shape[1]
    npad = ((n + RB - 1) // RB) * RB

    xp = jnp.pad(x, ((0, npad - n), (0, 0)))
    batch2d = jnp.pad(batch, (0, npad - n), constant_values=127)[:, None]
    b1r = b1[None, :]
    b2r = b2[None, :]
    b3r = b3[None, :]
    wlp = jnp.pad(Wl, ((0, 0), (0, 128 - c)))
    blp = jnp.pad(bl, (0, 128 - c))[None, :]

    chunk = NW * PF * EB * 2
    ep = ((e + chunk - 1) // chunk) * chunk
    pad = ep - e
    pidx = jnp.arange(pad, dtype=edge_index.dtype)
    srcp = jnp.concatenate([edge_index[0], pidx % n])
    dstp = jnp.concatenate([edge_index[1], n + pidx % (npad - n)])
    ei5 = jnp.concatenate(
        [srcp.reshape(NW, -1, PF, 1, EB), dstp.reshape(NW, -1, PF, 1, EB)],
        axis=3)
    dst3 = dstp.reshape(NW, -1, EB)

    degp = _sc_deg(dst3, npad)
    dinv, m1, r = _tc0(degp, xp, W1, batch2d)
    p1 = _sc_agg(m1, ei5)
    m2 = _tc_layer(p1, m1, dinv, b1r, W2)
    p2 = _sc_agg(m2, ei5)
    m3 = _tc_layer(p2, m2, dinv, b2r, W3)
    cp = _sc_agg(r, ei5, swap=True)
    out = _tc_final(cp, m3, dinv, b3r, batch2d, wlp, blp)
    return out[:, :c]


# speedup vs baseline: 1.1376x; 1.0028x over previous
import functools

import jax
import jax.numpy as jnp
from jax import lax
from jax.experimental import pallas as pl
from jax.experimental.pallas import tpu as pltpu
from jax.experimental.pallas import tpu_sc as plsc

NC = 2
NS = 16
NW = NC * NS
LANES = 16
EB = 128
PF = 4
RB = 1280
F32 = jnp.float32
HIGH = lax.Precision.HIGHEST


def _mesh():
    return plsc.VectorSubcoreMesh(
        core_axis_name="c", subcore_axis_name="s", num_cores=NC, num_subcores=NS
    )


def _sc_deg_body(dst3_hbm, out_hbm, idx_v, deg_v):
    c = lax.axis_index("c")
    s = lax.axis_index("s")
    wid = c * NS + s
    npad = deg_v.shape[0]
    nblk = dst3_hbm.shape[1]

    zeros16 = jnp.zeros((LANES,), F32)
    def zero_body(i, carry):
        deg_v[pl.ds(i * LANES, LANES)] = zeros16
        return carry
    lax.fori_loop(0, npad // LANES, zero_body, 0)

    pltpu.sync_copy(dst3_hbm.at[wid], idx_v)
    ones16 = jnp.ones((LANES,), F32)

    def body(j, carry):
        for k in range(EB // LANES):
            d = idx_v[j, pl.ds(k * LANES, LANES)]
            plsc.addupdate_scatter(deg_v, [d], ones16)
        return carry
    lax.fori_loop(0, nblk, body, 0)

    pltpu.sync_copy(deg_v, out_hbm.at[wid])


def _sc_agg_body(m_hbm, ei5_hbm, out_hbm, pa_v, pb_v, rows_a, rows_b,
                 sem_pa, sem_pb, sem_a0, sem_b0, acc_sh, *, gi=0, si=1):
    c = lax.axis_index("c")
    s = lax.axis_index("s")
    wid = c * NS + s
    npad = m_hbm.shape[0]
    width = m_hbm.shape[1]
    npanel = ei5_hbm.shape[1]
    rpt = npad // NS
    row0 = s * rpt

    zeros16 = jnp.zeros((LANES,), F32)
    def zero_body(i, carry):
        for k in range(width // LANES):
            rows_a[i, pl.ds(k * LANES, LANES)] = zeros16
        return carry
    lax.fori_loop(0, EB, zero_body, 0)
    for q in range(rpt // EB):
        pltpu.async_copy(rows_a, acc_sh.at[pl.ds(row0 + q * EB, EB)], sem_a0)
    for q in range(rpt // EB):
        pltpu.make_async_copy(rows_a, acc_sh.at[pl.ds(row0 + q * EB, EB)],
                              sem_a0).wait()
    plsc.subcore_barrier()

    slots = (rows_a, rows_b)
    sems = (sem_a0, sem_b0)

    def start_gather(panel, b, sl):
        pltpu.async_copy(m_hbm.at[panel.at[b, gi]], slots[sl], sems[sl])

    def wait_scat(panel, b, sl):
        pltpu.make_async_copy(m_hbm.at[panel.at[b, gi]], slots[sl],
                              sems[sl]).wait()
        pltpu.sync_copy(slots[sl], acc_sh.at[panel.at[b, si]], add=True)

    def load_panel(p_idx, panel, sem):
        pltpu.async_copy(ei5_hbm.at[wid, p_idx], panel, sem)

    def wait_panel(panel, sem):
        pltpu.make_async_copy(ei5_hbm.at[wid, 0], panel, sem).wait()

    pltpu.sync_copy(ei5_hbm.at[wid, 0], pa_v)
    start_gather(pa_v, 0, 0)
    load_panel(1, pb_v, sem_pb)

    npq = npanel // 2

    def body(q, carry):
        for b in range(PF):
            if b < PF - 1:
                start_gather(pa_v, b + 1, (b + 1) % 2)
            else:
                wait_panel(pb_v, sem_pb)
                start_gather(pb_v, 0, 0)
            wait_scat(pa_v, b, b % 2)

        @pl.when(q < npq - 1)
        def _():
            load_panel(2 * q + 2, pa_v, sem_pa)

        for b in range(PF):
            if b < PF - 1:
                start_gather(pb_v, b + 1, (b + 1) % 2)
            else:
                @pl.when(q < npq - 1)
                def _():
                    wait_panel(pa_v, sem_pa)
                    start_gather(pa_v, 0, 0)
            wait_scat(pb_v, b, b % 2)

        @pl.when(q < npq - 1)
        def _():
            load_panel(2 * q + 3, pb_v, sem_pb)
        return carry
    lax.fori_loop(0, npq, body, 0)
    plsc.subcore_barrier()

    pltpu.sync_copy(acc_sh.at[pl.ds(row0, rpt)], out_hbm.at[c, pl.ds(row0, rpt)])


def _sc_deg(dst3, npad):
    nblk = dst3.shape[1]
    fn = pl.kernel(
        _sc_deg_body,
        out_type=jax.ShapeDtypeStruct((NW, npad), F32),
        mesh=_mesh(),
        compiler_params=pltpu.CompilerParams(needs_layout_passes=False),
        scratch_types=[
            pltpu.VMEM((nblk, EB), jnp.int32),
            pltpu.VMEM((npad,), F32),
        ],
    )
    return fn(dst3)


def _sc_agg(m, ei5, swap=False):
    npad, width = m.shape
    fn = pl.kernel(
        functools.partial(_sc_agg_body, gi=1 if swap else 0,
                          si=0 if swap else 1),
        out_type=jax.ShapeDtypeStruct((NC, npad, width), F32),
        mesh=_mesh(),
        compiler_params=pltpu.CompilerParams(needs_layout_passes=False,
                                             use_tc_tiling_on_sc=False),
        scratch_types=[
            pltpu.VMEM((PF, 2, EB), jnp.int32),
            pltpu.VMEM((PF, 2, EB), jnp.int32),
            pltpu.VMEM((EB, width), F32),
            pltpu.VMEM((EB, width), F32),
            pltpu.SemaphoreType.DMA,
            pltpu.SemaphoreType.DMA,
            pltpu.SemaphoreType.DMA,
            pltpu.SemaphoreType.DMA,
            pltpu.VMEM_SHARED((npad, width), F32),
        ],
    )
    return fn(m, ei5)


def _tc0_body(degp_ref, x_ref, w_ref, batch_ref, dinv_ref, m_ref, r_ref):
    deg = jnp.sum(degp_ref[...], axis=0) + 1.0
    dinv = lax.rsqrt(deg)[:, None]
    dinv_ref[...] = dinv
    t = jnp.dot(x_ref[...], w_ref[...], precision=HIGH,
                preferred_element_type=F32)
    m_ref[...] = t * dinv
    g = lax.broadcasted_iota(jnp.int32, (1, 64), 1)
    r_ref[...] = (batch_ref[...] == g).astype(F32) * dinv


def _tc_layer_body(p_ref, m_ref, dinv_ref, b_ref, w_ref, out_ref):
    dinv = dinv_ref[...]
    h = jnp.maximum((p_ref[0] + p_ref[1] + m_ref[...]) * dinv + b_ref[...],
                    0.0)
    t = jnp.dot(h, w_ref[...], precision=HIGH, preferred_element_type=F32)
    out_ref[...] = t * dinv


def _tc_final_body(cp_ref, m_ref, dinv_ref, b_ref, batch_ref, wl_ref, bl_ref,
                   out_ref, pool_s, agg_s, cnt_s):
    i = pl.program_id(0)

    @pl.when(i == 0)
    def _init():
        pool_s[...] = jnp.zeros_like(pool_s)
        agg_s[...] = jnp.zeros_like(agg_s)
        cnt_s[...] = jnp.zeros_like(cnt_s)

    dinv = dinv_ref[...]
    m3 = m_ref[...]
    h3self = m3 * dinv
    g = lax.broadcasted_iota(jnp.int32, (1, 128), 1)
    oh = (batch_ref[...] == g).astype(F32)
    pool_s[...] += lax.dot_general(oh, h3self, (((0,), (0,)), ((), ())),
                                   precision=HIGH, preferred_element_type=F32)
    cb = cp_ref[0] + cp_ref[1]
    agg_s[...] += lax.dot_general(cb, m3, (((0,), (0,)), ((), ())),
                                  precision=HIGH, preferred_element_type=F32)
    cnt_s[...] += jnp.sum(oh, axis=0)[None, :]

    @pl.when(i == pl.num_programs(0) - 1)
    def _fin():
        cnt = jnp.maximum(cnt_s[...], 1.0)
        c64 = cnt[0, :64][:, None]
        total = pool_s[...][:64] + agg_s[...] + c64 * b_ref[...]
        pooled = total / c64
        out_ref[...] = jnp.dot(pooled, wl_ref[...], precision=HIGH,
                               preferred_element_type=F32) + bl_ref[...]


def _tc0(degp, xp, w1, batch2d):
    npad, f = xp.shape
    h = w1.shape[1]
    return pl.pallas_call(
        _tc0_body,
        grid=(npad // RB,),
        in_specs=[
            pl.BlockSpec((NW, RB), lambda i: (0, i)),
            pl.BlockSpec((RB, f), lambda i: (i, 0)),
            pl.BlockSpec((f, h), lambda i: (0, 0)),
            pl.BlockSpec((RB, 1), lambda i: (i, 0)),
        ],
        out_specs=[
            pl.BlockSpec((RB, 1), lambda i: (i, 0)),
            pl.BlockSpec((RB, h), lambda i: (i, 0)),
            pl.BlockSpec((RB, 64), lambda i: (i, 0)),
        ],
        out_shape=[
            jax.ShapeDtypeStruct((npad, 1), F32),
            jax.ShapeDtypeStruct((npad, h), F32),
            jax.ShapeDtypeStruct((npad, 64), F32),
        ],
    )(degp, xp, w1, batch2d)


def _tc_layer(p, m, dinv, b, w):
    npad, h = m.shape
    return pl.pallas_call(
        _tc_layer_body,
        grid=(npad // RB,),
        in_specs=[
            pl.BlockSpec((NC, RB, h), lambda i: (0, i, 0)),
            pl.BlockSpec((RB, h), lambda i: (i, 0)),
            pl.BlockSpec((RB, 1), lambda i: (i, 0)),
            pl.BlockSpec((1, h), lambda i: (0, 0)),
            pl.BlockSpec((h, h), lambda i: (0, 0)),
        ],
        out_specs=pl.BlockSpec((RB, h), lambda i: (i, 0)),
        out_shape=jax.ShapeDtypeStruct((npad, h), F32),
    )(p, m, dinv, b, w)


def _tc_final(cp, m, dinv, b, batch2d, wlp, blp):
    npad, h = m.shape
    return pl.pallas_call(
        _tc_final_body,
        grid=(npad // RB,),
        in_specs=[
            pl.BlockSpec((NC, RB, 64), lambda i: (0, i, 0)),
            pl.BlockSpec((RB, h), lambda i: (i, 0)),
            pl.BlockSpec((RB, 1), lambda i: (i, 0)),
            pl.BlockSpec((1, h), lambda i: (0, 0)),
            pl.BlockSpec((RB, 1), lambda i: (i, 0)),
            pl.BlockSpec((h, 128), lambda i: (0, 0)),
            pl.BlockSpec((1, 128), lambda i: (0, 0)),
        ],
        out_specs=pl.BlockSpec((64, 128), lambda i: (0, 0)),
        out_shape=jax.ShapeDtypeStruct((64, 128), F32),
        scratch_shapes=[
            pltpu.VMEM((128, 128), F32),
            pltpu.VMEM((64, 128), F32),
            pltpu.VMEM((1, 128), F32),
        ],
    )(cp, m, dinv, b, batch2d, wlp, blp)


def kernel(x, edge_index, batch, W1, b1, W2, b2, W3, b3, Wl, bl):
    n, f = x.shape
    h = W1.shape[1]
    c = Wl.shape[1]
    e = edge_index.shape[1]
    npad = ((n + RB - 1) // RB) * RB

    xp = jnp.pad(x, ((0, npad - n), (0, 0)))
    batch2d = jnp.pad(batch, (0, npad - n), constant_values=127)[:, None]
    b1r = b1[None, :]
    b2r = b2[None, :]
    b3r = b3[None, :]
    wlp = jnp.pad(Wl, ((0, 0), (0, 128 - c)))
    blp = jnp.pad(bl, (0, 128 - c))[None, :]

    chunk = NW * PF * EB * 2
    ep = ((e + chunk - 1) // chunk) * chunk
    pad = ep - e
    pidx = jnp.arange(pad, dtype=edge_index.dtype)
    srcp = jnp.concatenate([edge_index[0], pidx % n])
    dstp = jnp.concatenate([edge_index[1], n + pidx % (npad - n)])
    ei5 = jnp.concatenate(
        [srcp.reshape(NW, -1, PF, 1, EB), dstp.reshape(NW, -1, PF, 1, EB)],
        axis=3)
    dst3 = dstp.reshape(NW, -1, EB)

    degp = _sc_deg(dst3, npad)
    dinv, m1, r = _tc0(degp, xp, W1, batch2d)
    p1 = _sc_agg(m1, ei5)
    cp = _sc_agg(r, ei5, swap=True)
    m2 = _tc_layer(p1, m1, dinv, b1r, W2)
    p2 = _sc_agg(m2, ei5)
    m3 = _tc_layer(p2, m2, dinv, b2r, W3)
    out = _tc_final(cp, m3, dinv, b3r, batch2d, wlp, blp)
    return out[:, :c]


# speedup vs baseline: 1.1467x; 1.0080x over previous
import functools

import jax
import jax.numpy as jnp
from jax import lax
from jax.experimental import pallas as pl
from jax.experimental.pallas import tpu as pltpu
from jax.experimental.pallas import tpu_sc as plsc

NC = 2
NS = 16
NW = NC * NS
LANES = 16
EB = 128
PF = 4
RB = 1280
F32 = jnp.float32
HIGH = lax.Precision.HIGHEST


def _mesh():
    return plsc.VectorSubcoreMesh(
        core_axis_name="c", subcore_axis_name="s", num_cores=NC, num_subcores=NS
    )


def _sc_deg_body(dst3_hbm, out_hbm, idx_v, deg_v, sem):
    c = lax.axis_index("c")
    s = lax.axis_index("s")
    wid = c * NS + s
    npad = deg_v.shape[0]
    nblk = dst3_hbm.shape[1]

    pltpu.async_copy(dst3_hbm.at[wid], idx_v, sem)

    zeros16 = jnp.zeros((LANES,), F32)
    def zero_body(i, carry):
        deg_v[pl.ds(i * LANES, LANES)] = zeros16
        return carry
    lax.fori_loop(0, npad // LANES, zero_body, 0)

    pltpu.make_async_copy(dst3_hbm.at[wid], idx_v, sem).wait()
    ones16 = jnp.ones((LANES,), F32)

    def body(j, carry):
        for k in range(EB // LANES):
            d = idx_v[j, pl.ds(k * LANES, LANES)]
            plsc.addupdate_scatter(deg_v, [d], ones16)
        return carry
    lax.fori_loop(0, nblk, body, 0)

    pltpu.sync_copy(deg_v, out_hbm.at[wid])


def _sc_agg_body(m_hbm, ei5_hbm, out_hbm, pa_v, pb_v, rows_a, rows_b,
                 sem_pa, sem_pb, sem_a0, sem_b0, acc_sh, *, gi=0, si=1):
    c = lax.axis_index("c")
    s = lax.axis_index("s")
    wid = c * NS + s
    npad = m_hbm.shape[0]
    width = m_hbm.shape[1]
    npanel = ei5_hbm.shape[1]
    rpt = npad // NS
    row0 = s * rpt

    pltpu.async_copy(ei5_hbm.at[wid, 0], pa_v, sem_pa)

    zeros16 = jnp.zeros((LANES,), F32)
    def zero_body(i, carry):
        for k in range(width // LANES):
            rows_a[i, pl.ds(k * LANES, LANES)] = zeros16
        return carry
    lax.fori_loop(0, EB, zero_body, 0)
    for q in range(rpt // EB):
        pltpu.async_copy(rows_a, acc_sh.at[pl.ds(row0 + q * EB, EB)], sem_a0)
    for q in range(rpt // EB):
        pltpu.make_async_copy(rows_a, acc_sh.at[pl.ds(row0 + q * EB, EB)],
                              sem_a0).wait()
    plsc.subcore_barrier()

    slots = (rows_a, rows_b)
    sems = (sem_a0, sem_b0)

    def start_gather(panel, b, sl):
        pltpu.async_copy(m_hbm.at[panel.at[b, gi]], slots[sl], sems[sl])

    def wait_scat(panel, b, sl):
        pltpu.make_async_copy(m_hbm.at[panel.at[b, gi]], slots[sl],
                              sems[sl]).wait()
        pltpu.sync_copy(slots[sl], acc_sh.at[panel.at[b, si]], add=True)

    def load_panel(p_idx, panel, sem):
        pltpu.async_copy(ei5_hbm.at[wid, p_idx], panel, sem)

    def wait_panel(panel, sem):
        pltpu.make_async_copy(ei5_hbm.at[wid, 0], panel, sem).wait()

    wait_panel(pa_v, sem_pa)
    start_gather(pa_v, 0, 0)
    load_panel(1, pb_v, sem_pb)

    npq = npanel // 2

    def body(q, carry):
        for b in range(PF):
            if b < PF - 1:
                start_gather(pa_v, b + 1, (b + 1) % 2)
            else:
                wait_panel(pb_v, sem_pb)
                start_gather(pb_v, 0, 0)
            wait_scat(pa_v, b, b % 2)

        @pl.when(q < npq - 1)
        def _():
            load_panel(2 * q + 2, pa_v, sem_pa)

        for b in range(PF):
            if b < PF - 1:
                start_gather(pb_v, b + 1, (b + 1) % 2)
            else:
                @pl.when(q < npq - 1)
                def _():
                    wait_panel(pa_v, sem_pa)
                    start_gather(pa_v, 0, 0)
            wait_scat(pb_v, b, b % 2)

        @pl.when(q < npq - 1)
        def _():
            load_panel(2 * q + 3, pb_v, sem_pb)
        return carry
    lax.fori_loop(0, npq, body, 0)
    plsc.subcore_barrier()

    pltpu.sync_copy(acc_sh.at[pl.ds(row0, rpt)], out_hbm.at[c, pl.ds(row0, rpt)])


def _sc_deg(dst3, npad):
    nblk = dst3.shape[1]
    fn = pl.kernel(
        _sc_deg_body,
        out_type=jax.ShapeDtypeStruct((NW, npad), F32),
        mesh=_mesh(),
        compiler_params=pltpu.CompilerParams(needs_layout_passes=False),
        scratch_types=[
            pltpu.VMEM((nblk, EB), jnp.int32),
            pltpu.VMEM((npad,), F32),
            pltpu.SemaphoreType.DMA,
        ],
    )
    return fn(dst3)


def _sc_agg(m, ei5, swap=False):
    npad, width = m.shape
    fn = pl.kernel(
        functools.partial(_sc_agg_body, gi=1 if swap else 0,
                          si=0 if swap else 1),
        out_type=jax.ShapeDtypeStruct((NC, npad, width), F32),
        mesh=_mesh(),
        compiler_params=pltpu.CompilerParams(needs_layout_passes=False,
                                             use_tc_tiling_on_sc=False),
        scratch_types=[
            pltpu.VMEM((PF, 2, EB), jnp.int32),
            pltpu.VMEM((PF, 2, EB), jnp.int32),
            pltpu.VMEM((EB, width), F32),
            pltpu.VMEM((EB, width), F32),
            pltpu.SemaphoreType.DMA,
            pltpu.SemaphoreType.DMA,
            pltpu.SemaphoreType.DMA,
            pltpu.SemaphoreType.DMA,
            pltpu.VMEM_SHARED((npad, width), F32),
        ],
    )
    return fn(m, ei5)


def _tc0_body(degp_ref, x_ref, w_ref, batch_ref, dinv_ref, m_ref, r_ref):
    deg = jnp.sum(degp_ref[...], axis=0) + 1.0
    dinv = lax.rsqrt(deg)[:, None]
    dinv_ref[...] = dinv
    t = jnp.dot(x_ref[...], w_ref[...], precision=HIGH,
                preferred_element_type=F32)
    m_ref[...] = t * dinv
    g = lax.broadcasted_iota(jnp.int32, (1, 64), 1)
    r_ref[...] = (batch_ref[...] == g).astype(F32) * dinv


def _tc_layer_body(p_ref, m_ref, dinv_ref, b_ref, w_ref, out_ref):
    dinv = dinv_ref[...]
    h = jnp.maximum((p_ref[0] + p_ref[1] + m_ref[...]) * dinv + b_ref[...],
                    0.0)
    t = jnp.dot(h, w_ref[...], precision=HIGH, preferred_element_type=F32)
    out_ref[...] = t * dinv


def _tc_final_body(cp_ref, m_ref, dinv_ref, b_ref, batch_ref, wl_ref, bl_ref,
                   out_ref, pool_s, agg_s, cnt_s):
    i = pl.program_id(0)

    @pl.when(i == 0)
    def _init():
        pool_s[...] = jnp.zeros_like(pool_s)
        agg_s[...] = jnp.zeros_like(agg_s)
        cnt_s[...] = jnp.zeros_like(cnt_s)

    dinv = dinv_ref[...]
    m3 = m_ref[...]
    h3self = m3 * dinv
    g = lax.broadcasted_iota(jnp.int32, (1, 128), 1)
    oh = (batch_ref[...] == g).astype(F32)
    pool_s[...] += lax.dot_general(oh, h3self, (((0,), (0,)), ((), ())),
                                   precision=HIGH, preferred_element_type=F32)
    cb = cp_ref[0] + cp_ref[1]
    agg_s[...] += lax.dot_general(cb, m3, (((0,), (0,)), ((), ())),
                                  precision=HIGH, preferred_element_type=F32)
    cnt_s[...] += jnp.sum(oh, axis=0)[None, :]

    @pl.when(i == pl.num_programs(0) - 1)
    def _fin():
        cnt = jnp.maximum(cnt_s[...], 1.0)
        c64 = cnt[0, :64][:, None]
        total = pool_s[...][:64] + agg_s[...] + c64 * b_ref[...]
        pooled = total / c64
        out_ref[...] = jnp.dot(pooled, wl_ref[...], precision=HIGH,
                               preferred_element_type=F32) + bl_ref[...]


def _tc0(degp, xp, w1, batch2d):
    npad, f = xp.shape
    h = w1.shape[1]
    return pl.pallas_call(
        _tc0_body,
        grid=(npad // RB,),
        in_specs=[
            pl.BlockSpec((NW, RB), lambda i: (0, i)),
            pl.BlockSpec((RB, f), lambda i: (i, 0)),
            pl.BlockSpec((f, h), lambda i: (0, 0)),
            pl.BlockSpec((RB, 1), lambda i: (i, 0)),
        ],
        out_specs=[
            pl.BlockSpec((RB, 1), lambda i: (i, 0)),
            pl.BlockSpec((RB, h), lambda i: (i, 0)),
            pl.BlockSpec((RB, 64), lambda i: (i, 0)),
        ],
        out_shape=[
            jax.ShapeDtypeStruct((npad, 1), F32),
            jax.ShapeDtypeStruct((npad, h), F32),
            jax.ShapeDtypeStruct((npad, 64), F32),
        ],
    )(degp, xp, w1, batch2d)


def _tc_layer(p, m, dinv, b, w):
    npad, h = m.shape
    return pl.pallas_call(
        _tc_layer_body,
        grid=(npad // RB,),
        in_specs=[
            pl.BlockSpec((NC, RB, h), lambda i: (0, i, 0)),
            pl.BlockSpec((RB, h), lambda i: (i, 0)),
            pl.BlockSpec((RB, 1), lambda i: (i, 0)),
            pl.BlockSpec((1, h), lambda i: (0, 0)),
            pl.BlockSpec((h, h), lambda i: (0, 0)),
        ],
        out_specs=pl.BlockSpec((RB, h), lambda i: (i, 0)),
        out_shape=jax.ShapeDtypeStruct((npad, h), F32),
    )(p, m, dinv, b, w)


def _tc_final(cp, m, dinv, b, batch2d, wlp, blp):
    npad, h = m.shape
    return pl.pallas_call(
        _tc_final_body,
        grid=(npad // RB,),
        in_specs=[
            pl.BlockSpec((NC, RB, 64), lambda i: (0, i, 0)),
            pl.BlockSpec((RB, h), lambda i: (i, 0)),
            pl.BlockSpec((RB, 1), lambda i: (i, 0)),
            pl.BlockSpec((1, h), lambda i: (0, 0)),
            pl.BlockSpec((RB, 1), lambda i: (i, 0)),
            pl.BlockSpec((h, 128), lambda i: (0, 0)),
            pl.BlockSpec((1, 128), lambda i: (0, 0)),
        ],
        out_specs=pl.BlockSpec((64, 128), lambda i: (0, 0)),
        out_shape=jax.ShapeDtypeStruct((64, 128), F32),
        scratch_shapes=[
            pltpu.VMEM((128, 128), F32),
            pltpu.VMEM((64, 128), F32),
            pltpu.VMEM((1, 128), F32),
        ],
    )(cp, m, dinv, b, batch2d, wlp, blp)


def kernel(x, edge_index, batch, W1, b1, W2, b2, W3, b3, Wl, bl):
    n, f = x.shape
    h = W1.shape[1]
    c = Wl.shape[1]
    e = edge_index.shape[1]
    npad = ((n + RB - 1) // RB) * RB

    xp = jnp.pad(x, ((0, npad - n), (0, 0)))
    batch2d = jnp.pad(batch, (0, npad - n), constant_values=127)[:, None]
    b1r = b1[None, :]
    b2r = b2[None, :]
    b3r = b3[None, :]
    wlp = jnp.pad(Wl, ((0, 0), (0, 128 - c)))
    blp = jnp.pad(bl, (0, 128 - c))[None, :]

    chunk = NW * PF * EB * 2
    ep = ((e + chunk - 1) // chunk) * chunk
    pad = ep - e
    pidx = jnp.arange(pad, dtype=edge_index.dtype)
    srcp = jnp.concatenate([edge_index[0], pidx % n])
    dstp = jnp.concatenate([edge_index[1], n + pidx % (npad - n)])
    ei5 = jnp.concatenate(
        [srcp.reshape(NW, -1, PF, 1, EB), dstp.reshape(NW, -1, PF, 1, EB)],
        axis=3)
    dst3 = dstp.reshape(NW, -1, EB)

    degp = _sc_deg(dst3, npad)
    dinv, m1, r = _tc0(degp, xp, W1, batch2d)
    p1 = _sc_agg(m1, ei5)
    cp = _sc_agg(r, ei5, swap=True)
    m2 = _tc_layer(p1, m1, dinv, b1r, W2)
    p2 = _sc_agg(m2, ei5)
    m3 = _tc_layer(p2, m2, dinv, b2r, W3)
    out = _tc_final(cp, m3, dinv, b3r, batch2d, wlp, blp)
    return out[:, :c]
